# baseline (speedup 1.0000x reference)
import functools

import jax
import jax.numpy as jnp
from jax import lax
from jax.experimental import pallas as pl
from jax.experimental.pallas import tpu as pltpu
from jax.experimental.pallas import tpu_sc as plsc

HIDDEN = 64
BATCH = 16384
HIST = 50
VOCAB = 1000000
NPAIR = VOCAB // 2
NC, NS = 2, 16
NW = NC * NS
BCOL_W = BATCH // NW
N_BCHUNK = BCOL_W // 128

_mesh = plsc.VectorSubcoreMesh(core_axis_name="c", subcore_axis_name="s")


@functools.partial(
    pl.kernel,
    mesh=_mesh,
    out_type=jax.ShapeDtypeStruct((HIST, HIDDEN, BATCH), jnp.float32),
    scratch_types=[
        pltpu.VMEM((HIST, 128), jnp.int32),
        pltpu.VMEM((HIST, 128), jnp.int32),
        pltpu.VMEM((128, 128), jnp.float32),
        pltpu.VMEM((128, 128), jnp.float32),
        pltpu.VMEM((HIDDEN, 128), jnp.float32),
        pltpu.VMEM((HIDDEN, 128), jnp.float32),
        pltpu.SemaphoreType.DMA,
        pltpu.SemaphoreType.DMA,
        pltpu.SemaphoreType.DMA,
        pltpu.SemaphoreType.DMA,
    ],
    compiler_params=pltpu.CompilerParams(needs_layout_passes=False),
)
def _gather_k(idsT_hbm, pairs_hbm, out_hbm,
              ibuf, pbuf, G0, G1, T0, T1, gA, gB, sA, sB):
    wid = lax.axis_index("s") * NC + lax.axis_index("c")
    lane = lax.iota(jnp.int32, 16)

    def g_copy(h, gbuf, sem):
        return pltpu.make_async_copy(pairs_hbm.at[pbuf.at[h]], gbuf, sem)

    def s_copy(h, tbuf, b0, sem):
        return pltpu.make_async_copy(
            tbuf, out_hbm.at[h].at[:, pl.ds(b0, 128)], sem)

    def transpose_h(h, gbuf, tbuf):
        rows = [l0 * 16 + lane for l0 in range(8)]
        bases = [(ibuf[h, pl.ds(l0 * 16, 16)] & 1) * 64 for l0 in range(8)]

        def do_e(eo, ecarry):
            e0 = eo * 4
            vals = []
            for ei in range(4):
                for l0 in range(8):
                    vals.append(plsc.load_gather(
                        gbuf, [rows[l0], bases[l0] + (e0 + ei)]))
            i = 0
            for ei in range(4):
                for l0 in range(8):
                    tbuf[e0 + ei, pl.ds(l0 * 16, 16)] = vals[i]
                    i += 1
            return ecarry

        lax.fori_loop(0, HIDDEN // 4, do_e, 0)

    for c in range(N_BCHUNK):
        b0 = wid * BCOL_W + c * 128
        pltpu.sync_copy(idsT_hbm.at[:, pl.ds(b0, 128)], ibuf)

        def make_pairs(h, carry):
            for l0 in range(8):
                v = ibuf[h, pl.ds(l0 * 16, 16)]
                pbuf[h, pl.ds(l0 * 16, 16)] = lax.shift_right_logical(v, 1)
            return carry

        lax.fori_loop(0, HIST, make_pairs, 0)

        g_copy(0, G0, gA).start()

        def body(p, carry):
            h0 = 2 * p
            g_copy(h0, G0, gA).wait()
            g_copy(h0 + 1, G1, gB).start()

            @pl.when(p > 0)
            def _():
                s_copy(h0 - 2, T0, b0, sA).wait()

            transpose_h(h0, G0, T0)
            s_copy(h0, T0, b0, sA).start()

            g_copy(h0 + 1, G1, gB).wait()

            @pl.when(p < HIST // 2 - 1)
            def _():
                g_copy(h0 + 2, G0, gA).start()

            @pl.when(p > 0)
            def _():
                s_copy(h0 - 1, T1, b0, sB).wait()

            transpose_h(h0 + 1, G1, T1)
            s_copy(h0 + 1, T1, b0, sB).start()
            return carry

        lax.fori_loop(0, HIST // 2, body, 0)

        s_copy(HIST - 2, T0, b0, sA).wait()
        s_copy(HIST - 1, T1, b0, sB).wait()


def kernel(input_ids, embedding):
    pairs = embedding.reshape(NPAIR, 128)
    outT = _gather_k(input_ids.T, pairs)
    return outT.transpose(2, 0, 1)
